# baseline (speedup 1.0000x reference)
import functools

import jax
import jax.numpy as jnp
from jax import lax
from jax.experimental import pallas as pl
from jax.experimental.pallas import tpu as pltpu
from jax.experimental.pallas import tpu_sc as plsc

BATCH = 8
X_STEPS = 512
Y_STEPS = 4096
D_MODEL = 256

NC = 2
NS = 16
NW = NC * NS
LANES = 16

ROWS_PER_W = (BATCH * Y_STEPS) // NW
BLK = 128
NBLK = ROWS_PER_W // BLK
ZERO_ROW = BATCH * X_STEPS


def _sc_body_pipelined(table_hbm, idx_hbm, len_hbm, out_hbm,
                       idx_v, gidx_v, len_v, buf0, buf1, sem0, sem1):
    w = lax.axis_index("s") * NC + lax.axis_index("c")
    b = w // 4
    q = w % 4

    pltpu.sync_copy(idx_hbm.at[w], idx_v)
    pltpu.sync_copy(len_hbm.at[w], len_v)

    lenv = len_v[...]
    row_off = lax.broadcast(b * X_STEPS, (LANES,))
    zrow = jnp.full((LANES,), ZERO_ROW, jnp.int32)
    lane = lax.broadcasted_iota(jnp.int32, (LANES,), 0)

    for j in range(NBLK):
        for v in range(BLK // LANES):
            t = lane + (q * ROWS_PER_W + j * BLK + v * LANES)
            g = idx_v[j, pl.ds(v * LANES, LANES)] + row_off
            gidx_v[j, pl.ds(v * LANES, LANES)] = jnp.where(t < lenv, g, zrow)

    bufs = (buf0, buf1)
    sems = (sem0, sem1)
    copies = [None, None]
    copies[0] = pltpu.async_copy(table_hbm.at[gidx_v.at[0]], bufs[0], sems[0])
    for j in range(NBLK):
        s = j % 2
        copies[s].wait()
        if j + 1 < NBLK:
            s2 = (j + 1) % 2
            copies[s2] = pltpu.async_copy(
                table_hbm.at[gidx_v.at[j + 1]], bufs[s2], sems[s2])
        out_rows = out_hbm.at[pl.ds(w * ROWS_PER_W + j * BLK, BLK)]
        pltpu.sync_copy(bufs[s], out_rows)


@functools.cache
def _sc_call():
    mesh = plsc.VectorSubcoreMesh(
        core_axis_name="c", subcore_axis_name="s",
        num_cores=NC, num_subcores=NS)
    return pl.kernel(
        _sc_body_pipelined,
        out_type=jax.ShapeDtypeStruct((BATCH * Y_STEPS, D_MODEL), jnp.float32),
        mesh=mesh,
        scratch_types=[
            pltpu.VMEM((NBLK, BLK), jnp.int32),
            pltpu.VMEM((NBLK, BLK), jnp.int32),
            pltpu.VMEM((LANES,), jnp.int32),
            pltpu.VMEM((BLK, D_MODEL), jnp.float32),
            pltpu.VMEM((BLK, D_MODEL), jnp.float32),
            pltpu.SemaphoreType.DMA,
            pltpu.SemaphoreType.DMA,
        ],
    )


def kernel(phoneme_sequences, duration_indexes, output_length):
    table = jnp.concatenate(
        [phoneme_sequences.reshape(BATCH * X_STEPS, D_MODEL),
         jnp.zeros((8, D_MODEL), jnp.float32)], axis=0)
    idx3 = duration_indexes.reshape(NW, NBLK, BLK)
    len_rep = jnp.broadcast_to(
        jnp.repeat(output_length, NW // BATCH).astype(jnp.int32)[:, None],
        (NW, LANES))
    out = _sc_call()(table, idx3, len_rep)
    return out.reshape(BATCH, Y_STEPS, D_MODEL)


# speedup vs baseline: 8.7757x; 8.7757x over previous
import functools

import jax
import jax.numpy as jnp
from jax import lax
from jax.experimental import pallas as pl
from jax.experimental.pallas import tpu as pltpu
from jax.experimental.pallas import tpu_sc as plsc

BATCH = 8
X_STEPS = 512
Y_STEPS = 4096
D_MODEL = 256

NC = 2
NS = 16
NW = NC * NS
LANES = 16

ROWS_PER_W = (BATCH * Y_STEPS) // NW
BLK = 128
NBLK = ROWS_PER_W // BLK
ZERO_ROW = BATCH * X_STEPS
ZERO_PAD = 64


def _sc_body_pipelined(table_hbm, idx_hbm, len_hbm, out_hbm,
                       idx_v, gidx_v, len_v, buf0, buf1, sem0, sem1):
    w = lax.axis_index("s") * NC + lax.axis_index("c")
    b = w // 4
    q = w % 4

    pltpu.sync_copy(idx_hbm.at[w], idx_v)
    pltpu.sync_copy(len_hbm.at[w], len_v)

    lenv = len_v[...]
    row_off = lax.broadcast(b * X_STEPS, (LANES,))
    lane = lax.broadcasted_iota(jnp.int32, (LANES,), 0)

    for j in range(NBLK):
        for v in range(BLK // LANES):
            t = lane + (q * ROWS_PER_W + j * BLK + v * LANES)
            g = idx_v[j, pl.ds(v * LANES, LANES)] + row_off
            zrow = ZERO_ROW + (t & (ZERO_PAD - 1))
            gidx_v[j, pl.ds(v * LANES, LANES)] = jnp.where(t < lenv, g, zrow)

    bufs = (buf0, buf1)
    sems = (sem0, sem1)
    copies = [None, None]
    copies[0] = pltpu.async_copy(table_hbm.at[gidx_v.at[0]], bufs[0], sems[0])
    for j in range(NBLK):
        s = j % 2
        copies[s].wait()
        if j + 1 < NBLK:
            s2 = (j + 1) % 2
            copies[s2] = pltpu.async_copy(
                table_hbm.at[gidx_v.at[j + 1]], bufs[s2], sems[s2])
        out_rows = out_hbm.at[pl.ds(w * ROWS_PER_W + j * BLK, BLK)]
        pltpu.sync_copy(bufs[s], out_rows)


@functools.cache
def _sc_call():
    mesh = plsc.VectorSubcoreMesh(
        core_axis_name="c", subcore_axis_name="s",
        num_cores=NC, num_subcores=NS)
    return pl.kernel(
        _sc_body_pipelined,
        out_type=jax.ShapeDtypeStruct((BATCH * Y_STEPS, D_MODEL), jnp.float32),
        mesh=mesh,
        scratch_types=[
            pltpu.VMEM((NBLK, BLK), jnp.int32),
            pltpu.VMEM((NBLK, BLK), jnp.int32),
            pltpu.VMEM((LANES,), jnp.int32),
            pltpu.VMEM((BLK, D_MODEL), jnp.float32),
            pltpu.VMEM((BLK, D_MODEL), jnp.float32),
            pltpu.SemaphoreType.DMA,
            pltpu.SemaphoreType.DMA,
        ],
    )


def kernel(phoneme_sequences, duration_indexes, output_length):
    table = jnp.concatenate(
        [phoneme_sequences.reshape(BATCH * X_STEPS, D_MODEL),
         jnp.zeros((ZERO_PAD, D_MODEL), jnp.float32)], axis=0)
    idx3 = duration_indexes.reshape(NW, NBLK, BLK)
    len_rep = jnp.broadcast_to(
        jnp.repeat(output_length, NW // BATCH).astype(jnp.int32)[:, None],
        (NW, LANES))
    out = _sc_call()(table, idx3, len_rep)
    return out.reshape(BATCH, Y_STEPS, D_MODEL)


# speedup vs baseline: 9.5150x; 1.0842x over previous
import functools

import jax
import jax.numpy as jnp
from jax import lax
from jax.experimental import pallas as pl
from jax.experimental.pallas import tpu as pltpu
from jax.experimental.pallas import tpu_sc as plsc

BATCH = 8
X_STEPS = 512
Y_STEPS = 4096
D_MODEL = 256

NC = 2
NS = 16
NW = NC * NS
LANES = 16

ROWS_PER_W = (BATCH * Y_STEPS) // NW
BLK = 128
NBLK = ROWS_PER_W // BLK
VPB = BLK // LANES


def _zero_rows(buf, lo, hi):
    zv = jnp.zeros((LANES,), jnp.float32)

    def body(r, carry):
        for c in range(D_MODEL // LANES):
            buf[r, pl.ds(c * LANES, LANES)] = zv
        return carry

    lax.fori_loop(lo, hi, body, 0)


def _sc_body(table_hbm, idx_hbm, len_hbm, out_hbm,
             idx_v, gidx_v, len_v, buf0, buf1, zbuf,
             sem0, sem1, zsem):
    w = lax.axis_index("s") * NC + lax.axis_index("c")
    b = w // (NW // BATCH)
    q = w % (NW // BATCH)

    pltpu.sync_copy(idx_hbm.at[w], idx_v)
    pltpu.sync_copy(len_hbm.at[w], len_v)

    row_off = lax.broadcast(b * X_STEPS, (LANES,))

    for j in range(NBLK):
        for v in range(VPB):
            g = idx_v[j, pl.ds(v * LANES, LANES)] + row_off
            gidx_v[j, pl.ds(v * LANES, LANES)] = g

    n = jnp.clip(len_v[...][0] - q * ROWS_PER_W, 0, ROWS_PER_W)
    ng = (n + BLK - 1) // BLK
    rem = n - (ng - 1) * BLK

    _zero_rows(zbuf, 0, BLK)

    bufs = (buf0, buf1)
    sems = (sem0, sem1)

    def gather(j):
        return pltpu.async_copy(table_hbm.at[gidx_v.at[j]], bufs[j % 2],
                                sems[j % 2])

    @pl.when(0 < ng)
    def _():
        gather(0)

    for j in range(NBLK):
        out_rows = out_hbm.at[pl.ds(w * ROWS_PER_W + j * BLK, BLK)]

        @pl.when(j < ng)
        def _(j=j, out_rows=out_rows):
            pltpu.make_async_copy(table_hbm.at[gidx_v.at[j]], bufs[j % 2],
                                  sems[j % 2]).wait()

        if j + 1 < NBLK:
            @pl.when(j + 1 < ng)
            def _(j=j):
                gather(j + 1)

        @pl.when((j == ng - 1) & (rem < BLK))
        def _(j=j):
            _zero_rows(bufs[j % 2], rem, BLK)

        @pl.when(j < ng)
        def _(j=j, out_rows=out_rows):
            pltpu.sync_copy(bufs[j % 2], out_rows)

        @pl.when(j >= ng)
        def _(j=j, out_rows=out_rows):
            pltpu.async_copy(zbuf, out_rows, zsem)

    for j in range(NBLK):
        out_rows = out_hbm.at[pl.ds(w * ROWS_PER_W + j * BLK, BLK)]

        @pl.when(j >= ng)
        def _(j=j, out_rows=out_rows):
            pltpu.make_async_copy(zbuf, out_rows, zsem).wait()


@functools.cache
def _sc_call():
    mesh = plsc.VectorSubcoreMesh(
        core_axis_name="c", subcore_axis_name="s",
        num_cores=NC, num_subcores=NS)
    return pl.kernel(
        _sc_body,
        out_type=jax.ShapeDtypeStruct((BATCH * Y_STEPS, D_MODEL), jnp.float32),
        mesh=mesh,
        scratch_types=[
            pltpu.VMEM((NBLK, BLK), jnp.int32),
            pltpu.VMEM((NBLK, BLK), jnp.int32),
            pltpu.VMEM((LANES,), jnp.int32),
            pltpu.VMEM((BLK, D_MODEL), jnp.float32),
            pltpu.VMEM((BLK, D_MODEL), jnp.float32),
            pltpu.VMEM((BLK, D_MODEL), jnp.float32),
            pltpu.SemaphoreType.DMA,
            pltpu.SemaphoreType.DMA,
            pltpu.SemaphoreType.DMA,
        ],
    )


def kernel(phoneme_sequences, duration_indexes, output_length):
    table = phoneme_sequences.reshape(BATCH * X_STEPS, D_MODEL)
    idx3 = duration_indexes.reshape(NW, NBLK, BLK)
    len_rep = jnp.broadcast_to(
        jnp.repeat(output_length, NW // BATCH).astype(jnp.int32)[:, None],
        (NW, LANES))
    out = _sc_call()(table, idx3, len_rep)
    return out.reshape(BATCH, Y_STEPS, D_MODEL)


# speedup vs baseline: 10.8440x; 1.1397x over previous
import functools

import jax
import jax.numpy as jnp
from jax import lax
from jax.experimental import pallas as pl
from jax.experimental.pallas import tpu as pltpu
from jax.experimental.pallas import tpu_sc as plsc

BATCH = 8
X_STEPS = 512
Y_STEPS = 4096
D_MODEL = 256

NC = 2
NS = 16
NW = NC * NS
LANES = 16

ROWS_PER_W = (BATCH * Y_STEPS) // NW
BLK = 128
NBLK = ROWS_PER_W // BLK
VPB = BLK // LANES
NBUF = 3
ZROWS = 64


def _zero_rows(buf, lo, hi):
    zv = jnp.zeros((LANES,), jnp.float32)

    def body(r, carry):
        for c in range(D_MODEL // LANES):
            buf[r, pl.ds(c * LANES, LANES)] = zv
        return carry

    lax.fori_loop(lo, hi, body, 0)


def _sc_body(table_hbm, idx_hbm, len_hbm, out_hbm,
             idx_v, gidx_v, len_v, buf0, buf1, buf2, zbuf,
             gsem0, gsem1, gsem2, wsem0, wsem1, wsem2, zsem):
    w = lax.axis_index("s") * NC + lax.axis_index("c")
    b = w // (NW // BATCH)
    q = w % (NW // BATCH)

    pltpu.sync_copy(idx_hbm.at[w], idx_v)
    pltpu.sync_copy(len_hbm.at[w], len_v)

    row_off = lax.broadcast(b * X_STEPS, (LANES,))

    for j in range(NBLK):
        for v in range(VPB):
            g = idx_v[j, pl.ds(v * LANES, LANES)] + row_off
            gidx_v[j, pl.ds(v * LANES, LANES)] = g

    n = jnp.clip(len_v[...][0] - q * ROWS_PER_W, 0, ROWS_PER_W)
    ng = (n + BLK - 1) // BLK
    rem = n - (ng - 1) * BLK

    _zero_rows(zbuf, 0, ZROWS)

    for j in range(NBLK):
        @pl.when(j >= ng)
        def _(j=j):
            for h in range(BLK // ZROWS):
                pltpu.async_copy(
                    zbuf,
                    out_hbm.at[pl.ds(
                        w * ROWS_PER_W + j * BLK + h * ZROWS, ZROWS)],
                    zsem)

    bufs = (buf0, buf1, buf2)
    gsems = (gsem0, gsem1, gsem2)
    wsems = (wsem0, wsem1, wsem2)

    def gather(j):
        pltpu.async_copy(table_hbm.at[gidx_v.at[j]], bufs[j % NBUF],
                         gsems[j % NBUF])

    def out_at(j):
        return out_hbm.at[pl.ds(w * ROWS_PER_W + j * BLK, BLK)]

    for j in range(min(2, NBLK)):
        @pl.when(j < ng)
        def _(j=j):
            gather(j)

    for j in range(NBLK):
        @pl.when(j < ng)
        def _(j=j):
            pltpu.make_async_copy(table_hbm.at[gidx_v.at[j]], bufs[j % NBUF],
                                  gsems[j % NBUF]).wait()

        @pl.when((j == ng - 1) & (rem < BLK))
        def _(j=j):
            _zero_rows(bufs[j % NBUF], rem, BLK)

        @pl.when(j < ng)
        def _(j=j):
            pltpu.async_copy(bufs[j % NBUF], out_at(j), wsems[j % NBUF])

        if j + 2 < NBLK:
            if j >= 1:
                @pl.when(j + 2 < ng)
                def _(j=j):
                    pltpu.make_async_copy(bufs[(j - 1) % NBUF], out_at(j - 1),
                                          wsems[(j - 1) % NBUF]).wait()

            @pl.when(j + 2 < ng)
            def _(j=j):
                gather(j + 2)

    for k in range(NBLK):
        @pl.when((k < ng) & (k + NBUF >= ng))
        def _(k=k):
            pltpu.make_async_copy(bufs[k % NBUF], out_at(k),
                                  wsems[k % NBUF]).wait()

    for j in range(NBLK):
        @pl.when(j >= ng)
        def _(j=j):
            for h in range(BLK // ZROWS):
                pltpu.make_async_copy(
                    zbuf,
                    out_hbm.at[pl.ds(
                        w * ROWS_PER_W + j * BLK + h * ZROWS, ZROWS)],
                    zsem).wait()


@functools.cache
def _sc_call():
    mesh = plsc.VectorSubcoreMesh(
        core_axis_name="c", subcore_axis_name="s",
        num_cores=NC, num_subcores=NS)
    return pl.kernel(
        _sc_body,
        out_type=jax.ShapeDtypeStruct((BATCH * Y_STEPS, D_MODEL), jnp.float32),
        mesh=mesh,
        scratch_types=[
            pltpu.VMEM((NBLK, BLK), jnp.int32),
            pltpu.VMEM((NBLK, BLK), jnp.int32),
            pltpu.VMEM((LANES,), jnp.int32),
            pltpu.VMEM((BLK, D_MODEL), jnp.float32),
            pltpu.VMEM((BLK, D_MODEL), jnp.float32),
            pltpu.VMEM((BLK, D_MODEL), jnp.float32),
            pltpu.VMEM((ZROWS, D_MODEL), jnp.float32),
            pltpu.SemaphoreType.DMA,
            pltpu.SemaphoreType.DMA,
            pltpu.SemaphoreType.DMA,
            pltpu.SemaphoreType.DMA,
            pltpu.SemaphoreType.DMA,
            pltpu.SemaphoreType.DMA,
            pltpu.SemaphoreType.DMA,
        ],
    )


def kernel(phoneme_sequences, duration_indexes, output_length):
    table = phoneme_sequences.reshape(BATCH * X_STEPS, D_MODEL)
    idx3 = duration_indexes.reshape(NW, NBLK, BLK)
    len_rep = jnp.broadcast_to(
        jnp.repeat(output_length, NW // BATCH).astype(jnp.int32)[:, None],
        (NW, LANES))
    out = _sc_call()(table, idx3, len_rep)
    return out.reshape(BATCH, Y_STEPS, D_MODEL)


# speedup vs baseline: 10.8861x; 1.0039x over previous
import functools

import jax
import jax.numpy as jnp
from jax import lax
from jax.experimental import pallas as pl
from jax.experimental.pallas import tpu as pltpu
from jax.experimental.pallas import tpu_sc as plsc

BATCH = 8
X_STEPS = 512
Y_STEPS = 4096
D_MODEL = 256

NC = 2
NS = 16
NW = NC * NS
LANES = 16

ROWS_PER_W = (BATCH * Y_STEPS) // NW
BLK = 64
NBLK = ROWS_PER_W // BLK
VPB = BLK // LANES
NBUF = 6
PRIME = NBUF - 1
ZROWS = 64


def _zero_rows(buf, lo, hi):
    zv = jnp.zeros((LANES,), jnp.float32)

    def body(r, carry):
        for c in range(D_MODEL // LANES):
            buf[r, pl.ds(c * LANES, LANES)] = zv
        return carry

    lax.fori_loop(lo, hi, body, 0)


def _sc_body(table_hbm, idx_hbm, len_hbm, out_hbm, *scratch):
    it = iter(scratch)
    idx_v = next(it)
    gidx = tuple(next(it) for _ in range(NBLK))
    len_v = next(it)
    bufs = tuple(next(it) for _ in range(NBUF))
    zbuf = next(it)
    gsems = tuple(next(it) for _ in range(NBUF))
    wsems = tuple(next(it) for _ in range(NBUF))
    zsem = next(it)

    w = lax.axis_index("s") * NC + lax.axis_index("c")
    b = w // (NW // BATCH)
    q = w % (NW // BATCH)

    pltpu.sync_copy(idx_hbm.at[w], idx_v)
    pltpu.sync_copy(len_hbm.at[w], len_v)

    row_off = lax.broadcast(b * X_STEPS, (LANES,))

    for j in range(NBLK):
        for v in range(VPB):
            g = idx_v[j, pl.ds(v * LANES, LANES)] + row_off
            gidx[j][pl.ds(v * LANES, LANES)] = g

    n = jnp.clip(len_v[...][0] - q * ROWS_PER_W, 0, ROWS_PER_W)
    ng = (n + BLK - 1) // BLK
    rem = n - (ng - 1) * BLK

    _zero_rows(zbuf, 0, ZROWS)

    for j in range(NBLK):
        @pl.when(j >= ng)
        def _(j=j):
            for h in range(BLK // ZROWS):
                pltpu.async_copy(
                    zbuf,
                    out_hbm.at[pl.ds(
                        w * ROWS_PER_W + j * BLK + h * ZROWS, ZROWS)],
                    zsem)

    def gather(j):
        pltpu.async_copy(table_hbm.at[gidx[j]], bufs[j % NBUF],
                         gsems[j % NBUF])

    def out_at(j):
        return out_hbm.at[pl.ds(w * ROWS_PER_W + j * BLK, BLK)]

    for j in range(min(PRIME, NBLK)):
        @pl.when(j < ng)
        def _(j=j):
            gather(j)

    for j in range(NBLK):
        @pl.when(j < ng)
        def _(j=j):
            pltpu.make_async_copy(table_hbm.at[gidx[j]], bufs[j % NBUF],
                                  gsems[j % NBUF]).wait()

        @pl.when((j == ng - 1) & (rem < BLK))
        def _(j=j):
            _zero_rows(bufs[j % NBUF], rem, BLK)

        @pl.when(j < ng)
        def _(j=j):
            pltpu.async_copy(bufs[j % NBUF], out_at(j), wsems[j % NBUF])

        if j + PRIME < NBLK:
            if j + PRIME - NBUF >= 0:
                @pl.when(j + PRIME < ng)
                def _(j=j):
                    k = j + PRIME - NBUF
                    pltpu.make_async_copy(bufs[k % NBUF], out_at(k),
                                          wsems[k % NBUF]).wait()

            @pl.when(j + PRIME < ng)
            def _(j=j):
                gather(j + PRIME)

    for k in range(NBLK):
        @pl.when((k < ng) & (k + NBUF >= ng))
        def _(k=k):
            pltpu.make_async_copy(bufs[k % NBUF], out_at(k),
                                  wsems[k % NBUF]).wait()

    for j in range(NBLK):
        @pl.when(j >= ng)
        def _(j=j):
            for h in range(BLK // ZROWS):
                pltpu.make_async_copy(
                    zbuf,
                    out_hbm.at[pl.ds(
                        w * ROWS_PER_W + j * BLK + h * ZROWS, ZROWS)],
                    zsem).wait()


@functools.cache
def _sc_call():
    mesh = plsc.VectorSubcoreMesh(
        core_axis_name="c", subcore_axis_name="s",
        num_cores=NC, num_subcores=NS)
    return pl.kernel(
        _sc_body,
        out_type=jax.ShapeDtypeStruct((BATCH * Y_STEPS, D_MODEL), jnp.float32),
        mesh=mesh,
        scratch_types=[
            pltpu.VMEM((NBLK, BLK), jnp.int32),
            *[pltpu.VMEM((BLK,), jnp.int32) for _ in range(NBLK)],
            pltpu.VMEM((LANES,), jnp.int32),
            *[pltpu.VMEM((BLK, D_MODEL), jnp.float32) for _ in range(NBUF)],
            pltpu.VMEM((ZROWS, D_MODEL), jnp.float32),
            *[pltpu.SemaphoreType.DMA for _ in range(2 * NBUF + 1)],
        ],
    )


def kernel(phoneme_sequences, duration_indexes, output_length):
    table = phoneme_sequences.reshape(BATCH * X_STEPS, D_MODEL)
    idx3 = duration_indexes.reshape(NW, NBLK, BLK)
    len_rep = jnp.broadcast_to(
        jnp.repeat(output_length, NW // BATCH).astype(jnp.int32)[:, None],
        (NW, LANES))
    out = _sc_call()(table, idx3, len_rep)
    return out.reshape(BATCH, Y_STEPS, D_MODEL)


# speedup vs baseline: 11.0471x; 1.0148x over previous
import functools

import jax
import jax.numpy as jnp
from jax import lax
from jax.experimental import pallas as pl
from jax.experimental.pallas import tpu as pltpu
from jax.experimental.pallas import tpu_sc as plsc

BATCH = 8
X_STEPS = 512
Y_STEPS = 4096
D_MODEL = 256

NC = 2
NS = 16
NW = NC * NS
LANES = 16

BLK = 128
BPB = Y_STEPS // BLK
NSLOT = BATCH
VPB = BLK // LANES
NBUF = 3
PRIME = NBUF - 1
ROT = NW // BATCH
ZROWS = 64


def _zero_rows(buf, lo, hi):
    zv = jnp.zeros((LANES,), jnp.float32)

    def body(r, carry):
        for c in range(D_MODEL // LANES):
            buf[r, pl.ds(c * LANES, LANES)] = zv
        return carry

    lax.fori_loop(lo, hi, body, 0)


def _sc_body(table_hbm, idx_hbm, len_hbm, out_hbm, *scratch):
    it = iter(scratch)
    idx_v = next(it)
    gidx = tuple(next(it) for _ in range(NSLOT))
    len_v = next(it)
    bufs = tuple(next(it) for _ in range(NBUF))
    zbuf = next(it)
    gsems = tuple(next(it) for _ in range(NBUF))
    wsems = tuple(next(it) for _ in range(NBUF))
    zsem = next(it)

    w = lax.axis_index("s") * NC + lax.axis_index("c")

    pltpu.sync_copy(idx_hbm.at[w], idx_v)
    pltpu.sync_copy(len_hbm, len_v)
    lenvec = len_v[...]

    pos, valid, rem, out_off = [], [], [], []
    for j in range(NSLOT):
        p = jnp.mod(w - ROT * j, BPB)
        r = jnp.clip(lenvec[j] - p * BLK, 0, BLK)
        pos.append(p)
        rem.append(r)
        valid.append(r > 0)
        out_off.append(j * Y_STEPS + p * BLK)

    for j in range(NSLOT):
        roff = lax.broadcast(j * X_STEPS, (LANES,))
        for v in range(VPB):
            g = idx_v[j, pl.ds(v * LANES, LANES)] + roff
            gidx[j][pl.ds(v * LANES, LANES)] = g

    _zero_rows(zbuf, 0, ZROWS)

    for j in range(NSLOT):
        @pl.when(jnp.logical_not(valid[j]))
        def _(j=j):
            for h in range(BLK // ZROWS):
                pltpu.async_copy(
                    zbuf, out_hbm.at[pl.ds(out_off[j] + h * ZROWS, ZROWS)],
                    zsem)

    def gather(j):
        pltpu.async_copy(table_hbm.at[gidx[j]], bufs[j % NBUF],
                         gsems[j % NBUF])

    def out_at(j):
        return out_hbm.at[pl.ds(out_off[j], BLK)]

    for j in range(min(PRIME, NSLOT)):
        @pl.when(valid[j])
        def _(j=j):
            gather(j)

    for j in range(NSLOT):
        @pl.when(valid[j])
        def _(j=j):
            pltpu.make_async_copy(table_hbm.at[gidx[j]], bufs[j % NBUF],
                                  gsems[j % NBUF]).wait()

        @pl.when(valid[j] & (rem[j] < BLK))
        def _(j=j):
            _zero_rows(bufs[j % NBUF], rem[j], BLK)

        @pl.when(valid[j])
        def _(j=j):
            pltpu.async_copy(bufs[j % NBUF], out_at(j), wsems[j % NBUF])

        if j + PRIME < NSLOT:
            k = j + PRIME - NBUF
            if k >= 0:
                @pl.when(valid[j + PRIME] & valid[k])
                def _(j=j, k=k):
                    pltpu.make_async_copy(bufs[k % NBUF], out_at(k),
                                          wsems[k % NBUF]).wait()

            @pl.when(valid[j + PRIME])
            def _(j=j):
                gather(j + PRIME)

    for k in range(NSLOT):
        cond = valid[k]
        if k + NBUF < NSLOT:
            cond = cond & jnp.logical_not(valid[k + NBUF])

        @pl.when(cond)
        def _(k=k):
            pltpu.make_async_copy(bufs[k % NBUF], out_at(k),
                                  wsems[k % NBUF]).wait()

    for j in range(NSLOT):
        @pl.when(jnp.logical_not(valid[j]))
        def _(j=j):
            for h in range(BLK // ZROWS):
                pltpu.make_async_copy(
                    zbuf, out_hbm.at[pl.ds(out_off[j] + h * ZROWS, ZROWS)],
                    zsem).wait()


@functools.cache
def _sc_call():
    mesh = plsc.VectorSubcoreMesh(
        core_axis_name="c", subcore_axis_name="s",
        num_cores=NC, num_subcores=NS)
    return pl.kernel(
        _sc_body,
        out_type=jax.ShapeDtypeStruct((BATCH * Y_STEPS, D_MODEL), jnp.float32),
        mesh=mesh,
        scratch_types=[
            pltpu.VMEM((NSLOT, BLK), jnp.int32),
            *[pltpu.VMEM((BLK,), jnp.int32) for _ in range(NSLOT)],
            pltpu.VMEM((LANES,), jnp.int32),
            *[pltpu.VMEM((BLK, D_MODEL), jnp.float32) for _ in range(NBUF)],
            pltpu.VMEM((ZROWS, D_MODEL), jnp.float32),
            *[pltpu.SemaphoreType.DMA for _ in range(2 * NBUF + 1)],
        ],
    )


def kernel(phoneme_sequences, duration_indexes, output_length):
    table = phoneme_sequences.reshape(BATCH * X_STEPS, D_MODEL)
    d4 = duration_indexes.reshape(BATCH, BPB, BLK)
    rolled = jnp.stack(
        [jnp.roll(d4[k], ROT * k, axis=0) for k in range(BATCH)])
    idx3 = rolled.transpose(1, 0, 2)
    len16 = jnp.pad(output_length.astype(jnp.int32), (0, LANES - BATCH))
    out = _sc_call()(table, idx3, len16)
    return out.reshape(BATCH, Y_STEPS, D_MODEL)
